# Initial kernel scaffold; baseline (speedup 1.0000x reference)
import jax
import jax.numpy as jnp
from jax.experimental import pallas as pl


def kernel(x, edge_index, edge_features, W, b):
    raise NotImplementedError("write your pallas kernel here")


# speedup vs baseline: 2.7321x; 2.7321x over previous
import functools

import jax
import jax.numpy as jnp
from jax import lax
from jax.experimental import pallas as pl
from jax.experimental.pallas import tpu as pltpu
from jax.experimental.pallas import tpu_sc as plsc

_N_NODES = 10000
_N_EDGES = 320000
_D_FEAT = 128
_D_EDGE = 16
_D_OUT = 128
_D_HALF = _D_FEAT // 2
_D_AUG = 32

_NC = 2
_NS = 16
_G = 128
_EPT = 20480
_NGROUPS = _EPT // _G
_CHUNK = 8
_NCHUNKS = _NGROUPS // _CHUNK
_E_PAD = _NS * _EPT
_ROWS = 10112
_STRIPE = _ROWS // _NS


def _sc_segment_sums(x2, src2, dst_g, ef_t, zeros64, zeros16):
  mesh = plsc.VectorSubcoreMesh(core_axis_name="c", subcore_axis_name="s")

  @functools.partial(
      pl.kernel,
      out_type=(
          jax.ShapeDtypeStruct((_NC, _ROWS, _D_HALF), jnp.float32),
          jax.ShapeDtypeStruct((_NC, _ROWS, _D_AUG // 2), jnp.float32),
      ),
      mesh=mesh,
      compiler_params=pltpu.CompilerParams(use_tc_tiling_on_sc=False),
      scratch_types=[
          pltpu.VMEM_SHARED((_ROWS, _D_HALF), jnp.float32),
          pltpu.VMEM_SHARED((_ROWS, _D_AUG // 2), jnp.float32),
          pltpu.VMEM((_CHUNK, _G), jnp.int32),
          pltpu.VMEM((_CHUNK, _G), jnp.int32),
          pltpu.VMEM((_G, _D_HALF), jnp.float32),
          pltpu.VMEM((_G, _D_AUG // 2), jnp.float32),
          pltpu.SemaphoreType.DMA,
      ],
  )
  def k(x2_hbm, src_hbm, dst_hbm, ef_hbm, zeros64_hbm, zeros16_hbm,
        gp_hbm, ep_hbm,
        g_s, e_s, sidx, didx, rows_v, ef_v, sem):
    cid = lax.axis_index("c")
    sid = lax.axis_index("s")

    r0 = sid * _STRIPE
    pltpu.sync_copy(zeros64_hbm.at[pl.ds(r0, _STRIPE)],
                    g_s.at[pl.ds(r0, _STRIPE)])
    pltpu.sync_copy(zeros16_hbm.at[pl.ds(r0, _STRIPE)],
                    e_s.at[pl.ds(r0, _STRIPE)])
    plsc.subcore_barrier()

    ebase = sid * _EPT

    def body(g, carry):
      ch = g // _CHUNK
      j = g % _CHUNK

      @pl.when(j == 0)
      def _load_chunk():
        pltpu.sync_copy(src_hbm.at[cid, sid, ch], sidx)
        pltpu.sync_copy(dst_hbm.at[sid, ch], didx)

      pltpu.async_copy(x2_hbm.at[sidx.at[j]], rows_v, sem).wait()
      pltpu.sync_copy(ef_hbm.at[cid, pl.ds(ebase + g * _G, _G)], ef_v)
      pltpu.sync_copy(rows_v, g_s.at[didx.at[j]], add=True)
      pltpu.sync_copy(ef_v, e_s.at[didx.at[j]], add=True)
      return carry

    lax.fori_loop(0, _NGROUPS, body, 0)
    plsc.subcore_barrier()

    pltpu.sync_copy(g_s.at[pl.ds(r0, _STRIPE)],
                    gp_hbm.at[cid, pl.ds(r0, _STRIPE)])
    pltpu.sync_copy(e_s.at[pl.ds(r0, _STRIPE)],
                    ep_hbm.at[cid, pl.ds(r0, _STRIPE)])

  return k(x2, src2, dst_g, ef_t, zeros64, zeros16)


_TC_BLK = 1000


def _tc_body(gp_ref, ep_ref, w_ref, b_ref, o_ref):
  g = jnp.concatenate([gp_ref[0], gp_ref[1]], axis=-1)
  e = ep_ref[0]
  cnt = ep_ref[1][:, 0:1]
  acc = jnp.dot(g, w_ref[:_D_FEAT, :], preferred_element_type=jnp.float32)
  acc = acc + jnp.dot(e, w_ref[_D_FEAT:, :], preferred_element_type=jnp.float32)
  acc = acc + cnt * b_ref[...]
  o_ref[...] = acc / jnp.maximum(cnt, 1.0)


def _tc_combine(gp, ep, W, b2d):
  grid = _N_NODES // _TC_BLK
  return pl.pallas_call(
      _tc_body,
      grid=(grid,),
      in_specs=[
          pl.BlockSpec((_NC, _TC_BLK, _D_HALF), lambda i: (0, i, 0)),
          pl.BlockSpec((_NC, _TC_BLK, _D_AUG // 2), lambda i: (0, i, 0)),
          pl.BlockSpec((_D_FEAT + _D_EDGE, _D_OUT), lambda i: (0, 0)),
          pl.BlockSpec((1, _D_OUT), lambda i: (0, 0)),
      ],
      out_specs=pl.BlockSpec((_TC_BLK, _D_OUT), lambda i: (i, 0)),
      out_shape=jax.ShapeDtypeStruct((_N_NODES, _D_OUT), jnp.float32),
  )(gp, ep, W, b2d)


def kernel(x, edge_index, edge_features, W, b):
  ei = edge_index.astype(jnp.int32)
  pad = _E_PAD - _N_EDGES
  src = jnp.concatenate([ei[0], jnp.zeros((pad,), jnp.int32)])
  dst = jnp.concatenate([ei[1], jnp.full((pad,), _N_NODES, jnp.int32)])
  src2 = jnp.stack([src, src + _N_NODES]).reshape(_NC, _NS, _NCHUNKS, _CHUNK, _G)
  dst_g = dst.reshape(_NS, _NCHUNKS, _CHUNK, _G)
  x2 = x.reshape(_N_NODES, _NC, _D_HALF).transpose(1, 0, 2).reshape(
      _NC * _N_NODES, _D_HALF)
  ef_aug = jnp.concatenate(
      [edge_features,
       jnp.ones((_N_EDGES, 1), jnp.float32),
       jnp.zeros((_N_EDGES, _D_AUG - _D_EDGE - 1), jnp.float32)], axis=1)
  ef_aug = jnp.concatenate(
      [ef_aug, jnp.zeros((pad, _D_AUG), jnp.float32)], axis=0)
  ef_t = ef_aug.reshape(_E_PAD, _NC, _D_AUG // 2).transpose(1, 0, 2)
  zeros64 = jnp.zeros((_ROWS, _D_HALF), jnp.float32)
  zeros16 = jnp.zeros((_ROWS, _D_AUG // 2), jnp.float32)

  gp, ep = _sc_segment_sums(x2, src2, dst_g, ef_t, zeros64, zeros16)
  return _tc_combine(gp, ep, W, b.reshape(1, _D_OUT))


# speedup vs baseline: 3.4275x; 1.2545x over previous
import functools

import jax
import jax.numpy as jnp
from jax import lax
from jax.experimental import pallas as pl
from jax.experimental.pallas import tpu as pltpu
from jax.experimental.pallas import tpu_sc as plsc

_N_NODES = 10000
_N_EDGES = 320000
_D_FEAT = 128
_D_EDGE = 16
_D_OUT = 128
_D_HALF = _D_FEAT // 2
_D_AUG = 32

_NC = 2
_NS = 16
_G = 128
_EPT = 20480
_NGROUPS = _EPT // _G
_CHUNK = 8
_NCHUNKS = _NGROUPS // _CHUNK
_E_PAD = _NS * _EPT
_ROWS = 10112
_STRIPE = _ROWS // _NS


def _sc_segment_sums(x2, src2, dst_g, ef_t, zeros64, zeros16):
  mesh = plsc.VectorSubcoreMesh(core_axis_name="c", subcore_axis_name="s")

  @functools.partial(
      pl.kernel,
      out_type=(
          jax.ShapeDtypeStruct((_NC, _ROWS, _D_HALF), jnp.float32),
          jax.ShapeDtypeStruct((_NC, _ROWS, _D_AUG // 2), jnp.float32),
      ),
      mesh=mesh,
      compiler_params=pltpu.CompilerParams(use_tc_tiling_on_sc=False),
      scratch_types=[
          pltpu.VMEM_SHARED((_ROWS, _D_HALF), jnp.float32),
          pltpu.VMEM_SHARED((_ROWS, _D_AUG // 2), jnp.float32),
          pltpu.VMEM((_NGROUPS, _G), jnp.int32),
          pltpu.VMEM((_CHUNK, _G), jnp.int32),
          pltpu.VMEM((2, _G, _D_HALF), jnp.float32),
          pltpu.VMEM((2, _G, _D_AUG // 2), jnp.float32),
          pltpu.SemaphoreType.DMA,
          pltpu.SemaphoreType.DMA,
      ],
  )
  def k(x2_hbm, src_hbm, dst_hbm, ef_hbm, zeros64_hbm, zeros16_hbm,
        gp_hbm, ep_hbm,
        g_s, e_s, sidx, didx, rows_v, ef_v, semg, seme):
    cid = lax.axis_index("c")
    sid = lax.axis_index("s")

    r0 = sid * _STRIPE
    pltpu.sync_copy(zeros64_hbm.at[pl.ds(r0, _STRIPE)],
                    g_s.at[pl.ds(r0, _STRIPE)])
    pltpu.sync_copy(zeros16_hbm.at[pl.ds(r0, _STRIPE)],
                    e_s.at[pl.ds(r0, _STRIPE)])
    pltpu.sync_copy(src_hbm.at[cid, sid], sidx)
    plsc.subcore_barrier()

    ebase = sid * _EPT

    def _ef_src(g):
      return ef_hbm.at[cid, pl.ds(ebase + g * _G, _G)]

    pltpu.async_copy(x2_hbm.at[sidx.at[0]], rows_v.at[0], semg)
    pltpu.async_copy(_ef_src(0), ef_v.at[0], seme)

    def body(g, carry):
      ch = g // _CHUNK
      j = g % _CHUNK
      b = lax.rem(g, 2)

      @pl.when(j == 0)
      def _load_chunk():
        pltpu.sync_copy(dst_hbm.at[sid, ch], didx)

      pltpu.make_async_copy(x2_hbm.at[sidx.at[g]], rows_v.at[b], semg).wait()
      pltpu.make_async_copy(_ef_src(g), ef_v.at[b], seme).wait()

      @pl.when(g + 1 < _NGROUPS)
      def _prefetch():
        pltpu.async_copy(x2_hbm.at[sidx.at[g + 1]], rows_v.at[1 - b], semg)
        pltpu.async_copy(_ef_src(g + 1), ef_v.at[1 - b], seme)

      pltpu.sync_copy(rows_v.at[b], g_s.at[didx.at[j]], add=True)
      pltpu.sync_copy(ef_v.at[b], e_s.at[didx.at[j]], add=True)
      return carry

    lax.fori_loop(0, _NGROUPS, body, 0)
    plsc.subcore_barrier()

    pltpu.sync_copy(g_s.at[pl.ds(r0, _STRIPE)],
                    gp_hbm.at[cid, pl.ds(r0, _STRIPE)])
    pltpu.sync_copy(e_s.at[pl.ds(r0, _STRIPE)],
                    ep_hbm.at[cid, pl.ds(r0, _STRIPE)])

  return k(x2, src2, dst_g, ef_t, zeros64, zeros16)


_TC_BLK = 1000


def _tc_body(gp_ref, ep_ref, w_ref, b_ref, o_ref):
  g = jnp.concatenate([gp_ref[0], gp_ref[1]], axis=-1)
  e = ep_ref[0]
  cnt = ep_ref[1][:, 0:1]
  acc = jnp.dot(g, w_ref[:_D_FEAT, :], preferred_element_type=jnp.float32)
  acc = acc + jnp.dot(e, w_ref[_D_FEAT:, :], preferred_element_type=jnp.float32)
  acc = acc + cnt * b_ref[...]
  o_ref[...] = acc / jnp.maximum(cnt, 1.0)


def _tc_combine(gp, ep, W, b2d):
  grid = _N_NODES // _TC_BLK
  return pl.pallas_call(
      _tc_body,
      grid=(grid,),
      in_specs=[
          pl.BlockSpec((_NC, _TC_BLK, _D_HALF), lambda i: (0, i, 0)),
          pl.BlockSpec((_NC, _TC_BLK, _D_AUG // 2), lambda i: (0, i, 0)),
          pl.BlockSpec((_D_FEAT + _D_EDGE, _D_OUT), lambda i: (0, 0)),
          pl.BlockSpec((1, _D_OUT), lambda i: (0, 0)),
      ],
      out_specs=pl.BlockSpec((_TC_BLK, _D_OUT), lambda i: (i, 0)),
      out_shape=jax.ShapeDtypeStruct((_N_NODES, _D_OUT), jnp.float32),
  )(gp, ep, W, b2d)


def kernel(x, edge_index, edge_features, W, b):
  ei = edge_index.astype(jnp.int32)
  pad = _E_PAD - _N_EDGES
  src = jnp.concatenate([ei[0], jnp.zeros((pad,), jnp.int32)])
  dst = jnp.concatenate([ei[1], jnp.full((pad,), _N_NODES, jnp.int32)])
  src2 = jnp.stack([src, src + _N_NODES]).reshape(_NC, _NS, _NGROUPS, _G)
  dst_g = dst.reshape(_NS, _NCHUNKS, _CHUNK, _G)
  x2 = x.reshape(_N_NODES, _NC, _D_HALF).transpose(1, 0, 2).reshape(
      _NC * _N_NODES, _D_HALF)
  ef_aug = jnp.concatenate(
      [edge_features,
       jnp.ones((_N_EDGES, 1), jnp.float32),
       jnp.zeros((_N_EDGES, _D_AUG - _D_EDGE - 1), jnp.float32)], axis=1)
  ef_aug = jnp.concatenate(
      [ef_aug, jnp.zeros((pad, _D_AUG), jnp.float32)], axis=0)
  ef_t = ef_aug.reshape(_E_PAD, _NC, _D_AUG // 2).transpose(1, 0, 2)
  zeros64 = jnp.zeros((_ROWS, _D_HALF), jnp.float32)
  zeros16 = jnp.zeros((_ROWS, _D_AUG // 2), jnp.float32)

  gp, ep = _sc_segment_sums(x2, src2, dst_g, ef_t, zeros64, zeros16)
  return _tc_combine(gp, ep, W, b.reshape(1, _D_OUT))


# speedup vs baseline: 4.0940x; 1.1944x over previous
import functools

import jax
import jax.numpy as jnp
from jax import lax
from jax.experimental import pallas as pl
from jax.experimental.pallas import tpu as pltpu
from jax.experimental.pallas import tpu_sc as plsc

_N_NODES = 10000
_N_EDGES = 320000
_D_FEAT = 128
_D_EDGE = 16
_D_OUT = 128
_D_HALF = _D_FEAT // 2
_D_AUG = 32

_NC = 2
_NS = 16
_G = 128
_EPT = 20480
_NGROUPS = _EPT // _G
_CHUNK = 8
_NCHUNKS = _NGROUPS // _CHUNK
_E_PAD = _NS * _EPT
_ROWS = 10112
_STRIPE = _ROWS // _NS


def _sc_segment_sums(x2, src2, dst_g, ef, ones_blk, zeros64, zeros16):
  mesh = plsc.VectorSubcoreMesh(core_axis_name="c", subcore_axis_name="s")

  @functools.partial(
      pl.kernel,
      out_type=(
          jax.ShapeDtypeStruct((_NC, _ROWS, _D_HALF), jnp.float32),
          jax.ShapeDtypeStruct((_NC, _ROWS, _D_EDGE), jnp.float32),
      ),
      mesh=mesh,
      compiler_params=pltpu.CompilerParams(use_tc_tiling_on_sc=False),
      scratch_types=[
          pltpu.VMEM_SHARED((_ROWS, _D_HALF), jnp.float32),
          pltpu.VMEM_SHARED((_ROWS, _D_EDGE), jnp.float32),
          pltpu.VMEM((_NGROUPS, _G), jnp.int32),
          pltpu.VMEM((_CHUNK, _G), jnp.int32),
          pltpu.VMEM((2, _G, _D_HALF), jnp.float32),
          pltpu.VMEM((2, _G, _D_EDGE), jnp.float32),
          pltpu.VMEM((_G, _D_EDGE), jnp.float32),
          pltpu.SemaphoreType.DMA,
          pltpu.SemaphoreType.DMA,
      ],
  )
  def k(x2_hbm, src_hbm, dst_hbm, ef_hbm, ones_hbm, zeros64_hbm, zeros16_hbm,
        gp_hbm, ep_hbm,
        g_s, e_s, sidx, didx, rows_v, ef_v, ones_v, semg, seme):
    cid = lax.axis_index("c")
    sid = lax.axis_index("s")

    r0 = sid * _STRIPE
    pltpu.sync_copy(zeros64_hbm.at[pl.ds(r0, _STRIPE)],
                    g_s.at[pl.ds(r0, _STRIPE)])
    pltpu.sync_copy(zeros16_hbm.at[pl.ds(r0, _STRIPE)],
                    e_s.at[pl.ds(r0, _STRIPE)])
    pltpu.sync_copy(src_hbm.at[cid, sid], sidx)
    pltpu.sync_copy(ones_hbm, ones_v)
    plsc.subcore_barrier()

    ebase = sid * _EPT

    def _ef_src(g):
      return ef_hbm.at[pl.ds(jnp.minimum(ebase + g * _G, _N_EDGES - _G), _G)]

    pltpu.async_copy(x2_hbm.at[sidx.at[0]], rows_v.at[0], semg)

    @pl.when(cid == 0)
    def _ef_prologue():
      pltpu.async_copy(_ef_src(0), ef_v.at[0], seme)

    def body(g, carry):
      ch = g // _CHUNK
      j = g % _CHUNK
      b = lax.rem(g, 2)

      @pl.when(j == 0)
      def _load_chunk():
        pltpu.sync_copy(dst_hbm.at[sid, ch], didx)

      pltpu.make_async_copy(x2_hbm.at[sidx.at[g]], rows_v.at[b], semg).wait()

      @pl.when(cid == 0)
      def _ef_wait():
        pltpu.make_async_copy(_ef_src(g), ef_v.at[b], seme).wait()

      @pl.when(g + 1 < _NGROUPS)
      def _prefetch():
        pltpu.async_copy(x2_hbm.at[sidx.at[g + 1]], rows_v.at[1 - b], semg)

        @pl.when(cid == 0)
        def _ef_prefetch():
          pltpu.async_copy(_ef_src(g + 1), ef_v.at[1 - b], seme)

      pltpu.sync_copy(rows_v.at[b], g_s.at[didx.at[j]], add=True)

      @pl.when(cid == 0)
      def _ef_scatter():
        pltpu.sync_copy(ef_v.at[b], e_s.at[didx.at[j]], add=True)

      @pl.when(cid == 1)
      def _cnt_scatter():
        pltpu.sync_copy(ones_v, e_s.at[didx.at[j]], add=True)

      return carry

    lax.fori_loop(0, _NGROUPS, body, 0)
    plsc.subcore_barrier()

    pltpu.sync_copy(g_s.at[pl.ds(r0, _STRIPE)],
                    gp_hbm.at[cid, pl.ds(r0, _STRIPE)])
    pltpu.sync_copy(e_s.at[pl.ds(r0, _STRIPE)],
                    ep_hbm.at[cid, pl.ds(r0, _STRIPE)])

  return k(x2, src2, dst_g, ef, ones_blk, zeros64, zeros16)


_TC_BLK = 1000


def _tc_body(gp_ref, ep_ref, w_ref, b_ref, o_ref):
  g = jnp.concatenate([gp_ref[0], gp_ref[1]], axis=-1)
  e = ep_ref[0]
  cnt = ep_ref[1][:, 0:1]
  acc = jnp.dot(g, w_ref[:_D_FEAT, :], preferred_element_type=jnp.float32)
  acc = acc + jnp.dot(e, w_ref[_D_FEAT:, :], preferred_element_type=jnp.float32)
  acc = acc + cnt * b_ref[...]
  o_ref[...] = acc / jnp.maximum(cnt, 1.0)


def _tc_combine(gp, ep, W, b2d):
  grid = _N_NODES // _TC_BLK
  return pl.pallas_call(
      _tc_body,
      grid=(grid,),
      in_specs=[
          pl.BlockSpec((_NC, _TC_BLK, _D_HALF), lambda i: (0, i, 0)),
          pl.BlockSpec((_NC, _TC_BLK, _D_EDGE), lambda i: (0, i, 0)),
          pl.BlockSpec((_D_FEAT + _D_EDGE, _D_OUT), lambda i: (0, 0)),
          pl.BlockSpec((1, _D_OUT), lambda i: (0, 0)),
      ],
      out_specs=pl.BlockSpec((_TC_BLK, _D_OUT), lambda i: (i, 0)),
      out_shape=jax.ShapeDtypeStruct((_N_NODES, _D_OUT), jnp.float32),
  )(gp, ep, W, b2d)


def kernel(x, edge_index, edge_features, W, b):
  ei = edge_index.astype(jnp.int32)
  pad = _E_PAD - _N_EDGES
  src = jnp.concatenate([ei[0], jnp.zeros((pad,), jnp.int32)])
  dst = jnp.concatenate([ei[1], jnp.full((pad,), _N_NODES, jnp.int32)])
  src2 = jnp.stack([2 * src, 2 * src + 1]).reshape(_NC, _NS, _NGROUPS, _G)
  dst_g = dst.reshape(_NS, _NCHUNKS, _CHUNK, _G)
  x2 = x.reshape(_NC * _N_NODES, _D_HALF)
  ones_blk = jnp.concatenate(
      [jnp.ones((_G, 1), jnp.float32),
       jnp.zeros((_G, _D_EDGE - 1), jnp.float32)], axis=1)
  zeros64 = jnp.zeros((_ROWS, _D_HALF), jnp.float32)
  zeros16 = jnp.zeros((_ROWS, _D_EDGE), jnp.float32)

  gp, ep = _sc_segment_sums(x2, src2, dst_g, edge_features, ones_blk,
                            zeros64, zeros16)
  return _tc_combine(gp, ep, W, b.reshape(1, _D_OUT))
